# baseline (speedup 1.0000x reference)
import jax
import jax.numpy as jnp
from jax import lax
from jax.experimental import pallas as pl
from jax.experimental.pallas import tpu as pltpu, tpu_sc as plsc

_NC = 2
_NS = 16
_NW = _NC * _NS
_TR = 64
_XR = 32


def _make_sc_kernel(B, T, D, dtype):
    seq_per_w = T // _NW
    n_tc = seq_per_w // _TR
    n_h = _TR // _XR
    tlen = _TR * D
    xlen = _XR * D
    n_add = xlen // 16
    mesh = plsc.VectorSubcoreMesh(core_axis_name="c", subcore_axis_name="s")

    def body(x_hbm, tab_hbm, out_hbm, tbuf, xbuf, sem):
        wid = lax.axis_index("s") * _NC + lax.axis_index("c")
        seq0 = wid * seq_per_w

        @pl.loop(0, n_tc)
        def _chunk(c):
            t_off = (seq0 + c * _TR) * D
            pltpu.sync_copy(tab_hbm.at[pl.ds(t_off, tlen)], tbuf)

            @pl.loop(0, B * n_h)
            def _sub(s):
                b = s // n_h
                h = s % n_h
                x_off = b * T * D + t_off + h * xlen
                pltpu.sync_copy(x_hbm.at[pl.ds(x_off, xlen)], xbuf)

                @plsc.parallel_loop(0, n_add, unroll=8)
                def _add(j):
                    v = tbuf[pl.ds(h * xlen + j * 16, 16)]
                    plsc.addupdate(xbuf.at[pl.ds(j * 16, 16)], v)

                pltpu.sync_copy(xbuf, out_hbm.at[pl.ds(x_off, xlen)])

    return pl.kernel(
        body,
        out_type=jax.ShapeDtypeStruct((B * T * D,), dtype),
        mesh=mesh,
        scratch_types=[
            pltpu.VMEM((tlen,), dtype),
            pltpu.VMEM((xlen,), dtype),
            pltpu.SemaphoreType.DMA,
        ],
    )


def kernel(x, pos_table):
    B, T, D = x.shape
    x1 = x.reshape(B * T * D)
    tab1 = pos_table[:T].reshape(T * D)
    out = _make_sc_kernel(B, T, D, x.dtype)(x1, tab1)
    return out.reshape(B, T, D)


# speedup vs baseline: 1.1632x; 1.1632x over previous
import jax
import jax.numpy as jnp
from jax import lax
from jax.experimental import pallas as pl
from jax.experimental.pallas import tpu as pltpu, tpu_sc as plsc

_NC = 2
_NS = 16
_NW = _NC * _NS
_CR = 32


def _make_sc_kernel(B, T, D, dtype):
    seq_per_w = T // _NW
    n_tc = seq_per_w // _CR
    clen = _CR * D
    n_add = clen // 16
    mesh = plsc.VectorSubcoreMesh(core_axis_name="c", subcore_axis_name="s")

    def body(x_hbm, tab_hbm, out_hbm, tbuf, xb0, xb1, si0, si1, so0, so1):
        wid = lax.axis_index("s") * _NC + lax.axis_index("c")
        seq0 = wid * seq_per_w
        xbufs = (xb0, xb1)
        sin = (si0, si1)
        sout = (so0, so1)

        def x_off(c, b):
            return b * T * D + (seq0 + c * _CR) * D

        def load(c, b, p):
            return pltpu.async_copy(
                x_hbm.at[pl.ds(x_off(c, b), clen)], xbufs[p], sin[p])

        def wait_load(c, b, p):
            pltpu.make_async_copy(
                x_hbm.at[pl.ds(x_off(c, b), clen)], xbufs[p], sin[p]).wait()

        def store(c, b, p):
            return pltpu.async_copy(
                xbufs[p], out_hbm.at[pl.ds(x_off(c, b), clen)], sout[p])

        def wait_store(c, b, p):
            pltpu.make_async_copy(
                xbufs[p], out_hbm.at[pl.ds(x_off(c, b), clen)], sout[p]).wait()

        load(0, 0, 0)

        @pl.loop(0, n_tc)
        def _chunk(c):
            pltpu.sync_copy(tab_hbm.at[pl.ds((seq0 + c * _CR) * D, clen)], tbuf)
            for b in range(B):
                p = b % 2
                q = 1 - p
                if b == 0:
                    @pl.when(c > 0)
                    def _():
                        wait_store(c - 1, B - 1, q)
                else:
                    wait_store(c, b - 1, q)
                if b < B - 1:
                    load(c, b + 1, q)
                else:
                    @pl.when(c < n_tc - 1)
                    def _():
                        load(c + 1, 0, q)
                wait_load(c, b, p)

                @plsc.parallel_loop(0, n_add, unroll=8)
                def _add(j):
                    v = tbuf[pl.ds(j * 16, 16)]
                    plsc.addupdate(xbufs[p].at[pl.ds(j * 16, 16)], v)

                store(c, b, p)

        wait_store(n_tc - 1, B - 1, (B - 1) % 2)

    return pl.kernel(
        body,
        out_type=jax.ShapeDtypeStruct((B * T * D,), dtype),
        mesh=mesh,
        scratch_types=[
            pltpu.VMEM((clen,), dtype),
            pltpu.VMEM((clen,), dtype),
            pltpu.VMEM((clen,), dtype),
            pltpu.SemaphoreType.DMA,
            pltpu.SemaphoreType.DMA,
            pltpu.SemaphoreType.DMA,
            pltpu.SemaphoreType.DMA,
        ],
    )


def kernel(x, pos_table):
    B, T, D = x.shape
    x1 = x.reshape(B * T * D)
    tab1 = pos_table[:T].reshape(T * D)
    out = _make_sc_kernel(B, T, D, x.dtype)(x1, tab1)
    return out.reshape(B, T, D)


# speedup vs baseline: 2.7371x; 2.3530x over previous
import jax
import jax.numpy as jnp
from jax import lax
from jax.experimental import pallas as pl
from jax.experimental.pallas import tpu as pltpu, tpu_sc as plsc

_NC = 2
_NS = 16
_NW = _NC * _NS
_CR = 32


def _make_sc_kernel(B, T, D, dtype):
    seq_per_w = T // _NW
    n_tc = seq_per_w // _CR
    n_lane = D // 16
    mesh = plsc.VectorSubcoreMesh(core_axis_name="c", subcore_axis_name="s")

    def body(x_hbm, tab_hbm, out_hbm, tbuf, xb0, xb1, si0, si1, so0, so1):
        wid = lax.axis_index("s") * _NC + lax.axis_index("c")
        seq0 = wid * seq_per_w
        xbufs = (xb0, xb1)
        sin = (si0, si1)
        sout = (so0, so1)

        def row0(c, b):
            return b * T + seq0 + c * _CR

        def load(c, b, p):
            pltpu.async_copy(
                x_hbm.at[pl.ds(row0(c, b), _CR)], xbufs[p], sin[p])

        def wait_load(c, b, p):
            pltpu.make_async_copy(
                x_hbm.at[pl.ds(row0(c, b), _CR)], xbufs[p], sin[p]).wait()

        def store(c, b, p):
            pltpu.async_copy(
                xbufs[p], out_hbm.at[pl.ds(row0(c, b), _CR)], sout[p])

        def wait_store(c, b, p):
            pltpu.make_async_copy(
                xbufs[p], out_hbm.at[pl.ds(row0(c, b), _CR)], sout[p]).wait()

        load(0, 0, 0)

        @pl.loop(0, n_tc)
        def _chunk(c):
            pltpu.sync_copy(tab_hbm.at[pl.ds(seq0 + c * _CR, _CR)], tbuf)
            for b in range(B):
                p = b % 2
                q = 1 - p
                if b == 0:
                    @pl.when(c > 0)
                    def _():
                        wait_store(c - 1, B - 1, q)
                else:
                    wait_store(c, b - 1, q)
                if b < B - 1:
                    load(c, b + 1, q)
                else:
                    @pl.when(c < n_tc - 1)
                    def _():
                        load(c + 1, 0, q)
                wait_load(c, b, p)

                @plsc.parallel_loop(0, _CR, unroll=2)
                def _add(r):
                    for i in range(n_lane):
                        v = tbuf[r, pl.ds(i * 16, 16)]
                        plsc.addupdate(xbufs[p].at[r, pl.ds(i * 16, 16)], v)

                store(c, b, p)

        wait_store(n_tc - 1, B - 1, (B - 1) % 2)

    return pl.kernel(
        body,
        out_type=jax.ShapeDtypeStruct((B * T, D), dtype),
        mesh=mesh,
        scratch_types=[
            pltpu.VMEM((_CR, D), dtype),
            pltpu.VMEM((_CR, D), dtype),
            pltpu.VMEM((_CR, D), dtype),
            pltpu.SemaphoreType.DMA,
            pltpu.SemaphoreType.DMA,
            pltpu.SemaphoreType.DMA,
            pltpu.SemaphoreType.DMA,
        ],
    )


def kernel(x, pos_table):
    B, T, D = x.shape
    x2 = x.reshape(B * T, D)
    out = _make_sc_kernel(B, T, D, x.dtype)(x2, pos_table[:T])
    return out.reshape(B, T, D)
